# baseline (speedup 1.0000x reference)
import functools

import jax
import jax.numpy as jnp
from jax import lax
from jax.experimental import pallas as pl
from jax.experimental.pallas import tpu as pltpu
from jax.experimental.pallas import tpu_sc as plsc

B = 16
T = 16384
D = 128
NR = 5
NW = 32
ROWS_PER_W = T // NW
IDX_CH = 128
N_CH = ROWS_PER_W // IDX_CH
BLK = 2048
NBLK = T // BLK
NH = 2
HBLK = BLK // NH
NEG = -3e38

_NT = (((1,), (1,)), ((), ()))


def _sc_gather(users2d, nodes_v, utab, itab):
  mesh = plsc.VectorSubcoreMesh(core_axis_name="c", subcore_axis_name="s")

  @functools.partial(
      pl.kernel,
      mesh=mesh,
      out_type=(
          jax.ShapeDtypeStruct((T, D), jnp.float32),
          jax.ShapeDtypeStruct((B, D), jnp.float32),
      ),
      scratch_types=[
          pltpu.VMEM((N_CH, IDX_CH), jnp.int32),
          pltpu.VMEM((ROWS_PER_W, D), jnp.float32),
          pltpu.VMEM((B,), jnp.int32),
          pltpu.VMEM((B, D), jnp.float32),
          pltpu.SemaphoreType.DMA,
          pltpu.SemaphoreType.DMA,
      ],
  )
  def k(users_hbm, nodes_hbm, utab_hbm, itab_hbm, pt_hbm, qj_hbm,
        idx_v, rows_v, nidx_v, qrows_v, sem, qsem):
    wid = lax.axis_index("s") * 2 + lax.axis_index("c")
    pltpu.sync_copy(users_hbm.at[pl.ds(wid * N_CH, N_CH)], idx_v)
    copies = [
        pltpu.async_copy(utab_hbm.at[idx_v.at[c]],
                         rows_v.at[pl.ds(c * IDX_CH, IDX_CH)], sem)
        for c in range(N_CH)
    ]

    @pl.when(wid == 0)
    def _():
      pltpu.sync_copy(nodes_hbm, nidx_v)
      pltpu.async_copy(itab_hbm.at[nidx_v], qrows_v, qsem).wait()
      pltpu.sync_copy(qrows_v, qj_hbm)

    for cp in copies:
      cp.wait()
    pltpu.sync_copy(rows_v, pt_hbm.at[pl.ds(wid * ROWS_PER_W, ROWS_PER_W)])

  return k(users2d, nodes_v, utab, itab)


def _tc_body(pt_ref, rat_ref, segc_ref, segr_ref, qj_ref, r5_ref,
             g1_ref, g1b_ref, g2_ref, g2b_ref,
             a1_ref, a1b_ref, a2_ref, a2b_ref,
             a3_ref, a3b_ref, out_ref, macc, dacc, zacc, r1s, s1s):
  i = pl.program_id(0)
  f32 = jnp.float32

  @pl.when(i == 0)
  def _():
    macc[...] = jnp.full((B, 1), NEG, f32)
    dacc[...] = jnp.zeros((B, 1), f32)
    zacc[...] = jnp.zeros((B, D), f32)
    r1s[...] = jnp.zeros((8, D), f32)
    r1s[0:NR, :] = lax.dot_general(r5_ref[...], g1_ref[:, D:], _NT,
                                   preferred_element_type=f32) + g1b_ref[...]
    s1s[...] = lax.dot_general(qj_ref[...], a1_ref[:, D:], _NT,
                               preferred_element_type=f32) + a1b_ref[...]

  fjts, s_rows, sohTs, bms = [], [], [], []
  for hb in range(NH):
    lo, hi = hb * HBLK, (hb + 1) * HBLK
    pt = pt_ref[lo:hi, :]
    rat = rat_ref[lo:hi, :]
    segc = segc_ref[lo:hi, :]
    segr = segr_ref[0, 0:1, lo:hi]

    roh = (lax.broadcasted_iota(jnp.int32, (HBLK, 8), 1) == rat).astype(f32)
    soh = (lax.broadcasted_iota(jnp.int32, (HBLK, B), 1) == segc).astype(f32)
    sohT = lax.broadcasted_iota(jnp.int32, (B, HBLK), 0) == segr

    h = jnp.maximum(
        lax.dot_general(pt, g1_ref[:, :D], _NT, preferred_element_type=f32)
        + jnp.dot(roh, r1s[...], preferred_element_type=f32), 0.0)
    fjt = jnp.maximum(
        lax.dot_general(h, g2_ref[...], _NT, preferred_element_type=f32)
        + g2b_ref[...], 0.0)
    a = jnp.maximum(
        lax.dot_general(fjt, a1_ref[:, :D], _NT, preferred_element_type=f32)
        + jnp.dot(soh, s1s[...], preferred_element_type=f32), 0.0)
    a = jnp.maximum(
        lax.dot_general(a, a2_ref[...], _NT, preferred_element_type=f32)
        + a2b_ref[...], 0.0)
    s_row = lax.dot_general(a3_ref[...], a, _NT,
                            preferred_element_type=f32) + a3b_ref[0, 0]

    fjts.append(fjt)
    s_rows.append(s_row)
    sohTs.append(sohT)
    bms.append(jnp.max(jnp.where(sohT, s_row, NEG), axis=1, keepdims=True))

  m_old = macc[...]
  m_new = functools.reduce(jnp.maximum, bms, m_old)
  scale = jnp.exp(m_old - m_new)
  dsum = dacc[...] * scale
  zsum = zacc[...] * scale
  for hb in range(NH):
    m_tok = jnp.max(jnp.where(sohTs[hb], m_new, NEG), axis=0, keepdims=True)
    e_row = jnp.exp(s_rows[hb] - m_tok)
    w = jnp.where(sohTs[hb], e_row, 0.0)
    dsum = dsum + jnp.sum(w, axis=1, keepdims=True)
    zsum = zsum + jnp.dot(w, fjts[hb], preferred_element_type=f32)
  dacc[...] = dsum
  zacc[...] = zsum
  macc[...] = m_new

  @pl.when(i == NBLK - 1)
  def _():
    d = dacc[...]
    out_ref[...] = zacc[...] / jnp.where(d > 0, d, 1.0)


def _tc_call(pt, qj, flat_ratings, segment_ids, embed_r_w,
             g1_w, g1_b, g2_w, g2_b, a1_w, a1_b, a2_w, a2_b, a3_w, a3_b):
  full = lambda shape: pl.BlockSpec(shape, lambda i: (0,) * len(shape))
  return pl.pallas_call(
      _tc_body,
      grid=(NBLK,),
      in_specs=[
          pl.BlockSpec((BLK, D), lambda i: (i, 0)),
          pl.BlockSpec((BLK, 1), lambda i: (i, 0)),
          pl.BlockSpec((BLK, 1), lambda i: (i, 0)),
          pl.BlockSpec((1, 1, BLK), lambda i: (i, 0, 0)),
          full((B, D)),
          full((NR, D)),
          full((D, 2 * D)),
          full((1, D)),
          full((D, D)),
          full((1, D)),
          full((D, 2 * D)),
          full((1, D)),
          full((D, D)),
          full((1, D)),
          full((1, D)),
          full((1, 1)),
      ],
      out_specs=pl.BlockSpec((B, D), lambda i: (0, 0)),
      out_shape=jax.ShapeDtypeStruct((B, D), jnp.float32),
      scratch_shapes=[
          pltpu.VMEM((B, 1), jnp.float32),
          pltpu.VMEM((B, 1), jnp.float32),
          pltpu.VMEM((B, D), jnp.float32),
          pltpu.VMEM((8, D), jnp.float32),
          pltpu.VMEM((B, D), jnp.float32),
      ],
      compiler_params=pltpu.CompilerParams(
          dimension_semantics=("arbitrary",)),
  )(pt, flat_ratings.reshape(T, 1), segment_ids.reshape(T, 1),
    segment_ids.reshape(NBLK, 1, BLK), qj, embed_r_w,
    g1_w, g1_b.reshape(1, D), g2_w, g2_b.reshape(1, D),
    a1_w, a1_b.reshape(1, D), a2_w, a2_b.reshape(1, D),
    a3_w, a3_b.reshape(1, 1))


def kernel(nodes_v, flat_users, flat_ratings, segment_ids, embed_u_w,
           embed_i_w, embed_r_w, g1_w, g1_b, g2_w, g2_b, a1_w, a1_b,
           a2_w, a2_b, a3_w, a3_b):
  users2d = flat_users.reshape(T // IDX_CH, IDX_CH)
  pt, qj = _sc_gather(users2d, nodes_v, embed_u_w, embed_i_w)
  return _tc_call(pt, qj, flat_ratings, segment_ids, embed_r_w,
                  g1_w, g1_b, g2_w, g2_b, a1_w, a1_b, a2_w, a2_b, a3_w, a3_b)


# speedup vs baseline: 1.0670x; 1.0670x over previous
import functools

import jax
import jax.numpy as jnp
from jax import lax
from jax.experimental import pallas as pl
from jax.experimental.pallas import tpu as pltpu
from jax.experimental.pallas import tpu_sc as plsc

B = 16
T = 16384
D = 128
NR = 5
NW = 32
ROWS_PER_W = T // NW
IDX_CH = 128
N_CH = ROWS_PER_W // IDX_CH
BLK = 2048
NBLK = T // BLK
NH = 2
HBLK = BLK // NH
NEG = -3e38

_NT = (((1,), (1,)), ((), ()))


def _sc_gather(users2d, nodes_v, utab, itab):
  mesh = plsc.VectorSubcoreMesh(core_axis_name="c", subcore_axis_name="s")

  @functools.partial(
      pl.kernel,
      mesh=mesh,
      out_type=(
          jax.ShapeDtypeStruct((T, D), jnp.float32),
          jax.ShapeDtypeStruct((B, D), jnp.float32),
      ),
      scratch_types=[
          pltpu.VMEM((N_CH, IDX_CH), jnp.int32),
          pltpu.VMEM((ROWS_PER_W, D), jnp.float32),
          pltpu.VMEM((B,), jnp.int32),
          pltpu.VMEM((B, D), jnp.float32),
          pltpu.SemaphoreType.DMA,
          pltpu.SemaphoreType.DMA,
      ],
  )
  def k(users_hbm, nodes_hbm, utab_hbm, itab_hbm, pt_hbm, qj_hbm,
        idx_v, rows_v, nidx_v, qrows_v, sem, qsem):
    wid = lax.axis_index("s") * 2 + lax.axis_index("c")
    pltpu.sync_copy(users_hbm.at[pl.ds(wid * N_CH, N_CH)], idx_v)
    copies = [
        pltpu.async_copy(utab_hbm.at[idx_v.at[c]],
                         rows_v.at[pl.ds(c * IDX_CH, IDX_CH)], sem)
        for c in range(N_CH)
    ]

    @pl.when(wid == 0)
    def _():
      pltpu.sync_copy(nodes_hbm, nidx_v)
      pltpu.async_copy(itab_hbm.at[nidx_v], qrows_v, qsem).wait()
      pltpu.sync_copy(qrows_v, qj_hbm)

    for cp in copies:
      cp.wait()
    pltpu.sync_copy(rows_v, pt_hbm.at[pl.ds(wid * ROWS_PER_W, ROWS_PER_W)])

  return k(users2d, nodes_v, utab, itab)


def _tc_body(pt_ref, rat_ref, segc_ref, segr_ref, qj_ref, r5_ref,
             g1_ref, g1b_ref, g2_ref, g2b_ref,
             a1_ref, a1b_ref, a2_ref, a2b_ref,
             a3_ref, a3b_ref, out_ref, macc, dacc, zacc,
             r1s, s1s, g1abf, g2bf, a1abf, a2bf):
  i = pl.program_id(0)
  f32 = jnp.float32
  bf16 = jnp.bfloat16

  @pl.when(i == 0)
  def _():
    macc[...] = jnp.full((B, 1), NEG, f32)
    dacc[...] = jnp.zeros((B, 1), f32)
    zacc[...] = jnp.zeros((B, D), f32)
    r1 = (lax.dot_general(r5_ref[...], g1_ref[:, D:], _NT,
                          preferred_element_type=f32) + g1b_ref[...])
    r1s[...] = jnp.concatenate(
        [r1, jnp.zeros((8 - NR, D), f32)], axis=0).astype(bf16)
    s1s[...] = (lax.dot_general(qj_ref[...], a1_ref[:, D:], _NT,
                                preferred_element_type=f32)
                + a1b_ref[...]).astype(bf16)
    g1abf[...] = g1_ref[:, :D].astype(bf16)
    g2bf[...] = g2_ref[...].astype(bf16)
    a1abf[...] = a1_ref[:, :D].astype(bf16)
    a2bf[...] = a2_ref[...].astype(bf16)

  pt = pt_ref[...].astype(bf16)
  rat = rat_ref[...]
  segc = segc_ref[...]
  segr = segr_ref[0, 0:1, :]

  roh = (lax.broadcasted_iota(jnp.int32, (BLK, 8), 1) == rat).astype(bf16)
  soh = (lax.broadcasted_iota(jnp.int32, (BLK, B), 1) == segc).astype(bf16)
  sohT = lax.broadcasted_iota(jnp.int32, (B, BLK), 0) == segr

  h = jnp.maximum(
      lax.dot_general(pt, g1abf[...], _NT, preferred_element_type=f32)
      + jnp.dot(roh, r1s[...], preferred_element_type=f32), 0.0).astype(bf16)
  fjt = jnp.maximum(
      lax.dot_general(h, g2bf[...], _NT, preferred_element_type=f32)
      + g2b_ref[...], 0.0).astype(bf16)
  a = jnp.maximum(
      lax.dot_general(fjt, a1abf[...], _NT, preferred_element_type=f32)
      + jnp.dot(soh, s1s[...], preferred_element_type=f32), 0.0).astype(bf16)
  a = jnp.maximum(
      lax.dot_general(a, a2bf[...], _NT, preferred_element_type=f32)
      + a2b_ref[...], 0.0).astype(bf16)
  s_row = lax.dot_general(a3_ref[...].astype(bf16), a, _NT,
                          preferred_element_type=f32) + a3b_ref[0, 0]

  bm = jnp.max(jnp.where(sohT, s_row, NEG), axis=1, keepdims=True)
  m_old = macc[...]
  m_new = jnp.maximum(m_old, bm)
  scale = jnp.exp(m_old - m_new)
  m_tok = jnp.max(jnp.where(sohT, m_new, NEG), axis=0, keepdims=True)
  e_row = jnp.exp(s_row - m_tok)
  w = jnp.where(sohT, e_row, 0.0)
  dacc[...] = dacc[...] * scale + jnp.sum(w, axis=1, keepdims=True)
  zacc[...] = zacc[...] * scale + jnp.dot(w.astype(bf16), fjt,
                                          preferred_element_type=f32)
  macc[...] = m_new

  @pl.when(i == NBLK - 1)
  def _():
    d = dacc[...]
    out_ref[...] = zacc[...] / jnp.where(d > 0, d, 1.0)


def _tc_call(pt, qj, flat_ratings, segment_ids, embed_r_w,
             g1_w, g1_b, g2_w, g2_b, a1_w, a1_b, a2_w, a2_b, a3_w, a3_b):
  full = lambda shape: pl.BlockSpec(shape, lambda i: (0,) * len(shape))
  return pl.pallas_call(
      _tc_body,
      grid=(NBLK,),
      in_specs=[
          pl.BlockSpec((BLK, D), lambda i: (i, 0)),
          pl.BlockSpec((BLK, 1), lambda i: (i, 0)),
          pl.BlockSpec((BLK, 1), lambda i: (i, 0)),
          pl.BlockSpec((1, 1, BLK), lambda i: (i, 0, 0)),
          full((B, D)),
          full((NR, D)),
          full((D, 2 * D)),
          full((1, D)),
          full((D, D)),
          full((1, D)),
          full((D, 2 * D)),
          full((1, D)),
          full((D, D)),
          full((1, D)),
          full((1, D)),
          full((1, 1)),
      ],
      out_specs=pl.BlockSpec((B, D), lambda i: (0, 0)),
      out_shape=jax.ShapeDtypeStruct((B, D), jnp.float32),
      scratch_shapes=[
          pltpu.VMEM((B, 1), jnp.float32),
          pltpu.VMEM((B, 1), jnp.float32),
          pltpu.VMEM((B, D), jnp.float32),
          pltpu.VMEM((8, D), jnp.bfloat16),
          pltpu.VMEM((B, D), jnp.bfloat16),
          pltpu.VMEM((D, D), jnp.bfloat16),
          pltpu.VMEM((D, D), jnp.bfloat16),
          pltpu.VMEM((D, D), jnp.bfloat16),
          pltpu.VMEM((D, D), jnp.bfloat16),
      ],
      compiler_params=pltpu.CompilerParams(
          dimension_semantics=("arbitrary",)),
  )(pt, flat_ratings.reshape(T, 1), segment_ids.reshape(T, 1),
    segment_ids.reshape(NBLK, 1, BLK), qj, embed_r_w,
    g1_w, g1_b.reshape(1, D), g2_w, g2_b.reshape(1, D),
    a1_w, a1_b.reshape(1, D), a2_w, a2_b.reshape(1, D),
    a3_w, a3_b.reshape(1, 1))


def kernel(nodes_v, flat_users, flat_ratings, segment_ids, embed_u_w,
           embed_i_w, embed_r_w, g1_w, g1_b, g2_w, g2_b, a1_w, a1_b,
           a2_w, a2_b, a3_w, a3_b):
  users2d = flat_users.reshape(T // IDX_CH, IDX_CH)
  pt, qj = _sc_gather(users2d, nodes_v, embed_u_w, embed_i_w)
  return _tc_call(pt, qj, flat_ratings, segment_ids, embed_r_w,
                  g1_w, g1_b, g2_w, g2_b, a1_w, a1_b, a2_w, a2_b, a3_w, a3_b)


# speedup vs baseline: 1.1733x; 1.0996x over previous
import functools

import jax
import jax.numpy as jnp
from jax import lax
from jax.experimental import pallas as pl
from jax.experimental.pallas import tpu as pltpu
from jax.experimental.pallas import tpu_sc as plsc

B = 16
T = 16384
D = 128
NR = 5
NW = 32
ROWS_PER_W = T // NW
IDX_CH = 128
N_CH = ROWS_PER_W // IDX_CH
BLK = 2048
NBLK = T // BLK
NH = 2
HBLK = BLK // NH
NEG = -3e38

_NT = (((1,), (1,)), ((), ()))
_TN = (((0,), (0,)), ((), ()))


def _sc_gather(users2d, nodes_v, utab, itab):
  mesh = plsc.VectorSubcoreMesh(core_axis_name="c", subcore_axis_name="s")

  @functools.partial(
      pl.kernel,
      mesh=mesh,
      out_type=(
          jax.ShapeDtypeStruct((T, D), jnp.float32),
          jax.ShapeDtypeStruct((B, D), jnp.float32),
      ),
      scratch_types=[
          pltpu.VMEM((N_CH, IDX_CH), jnp.int32),
          pltpu.VMEM((ROWS_PER_W, D), jnp.float32),
          pltpu.VMEM((B,), jnp.int32),
          pltpu.VMEM((B, D), jnp.float32),
          pltpu.SemaphoreType.DMA,
          pltpu.SemaphoreType.DMA,
      ],
  )
  def k(users_hbm, nodes_hbm, utab_hbm, itab_hbm, pt_hbm, qj_hbm,
        idx_v, rows_v, nidx_v, qrows_v, sem, qsem):
    wid = lax.axis_index("s") * 2 + lax.axis_index("c")
    pltpu.sync_copy(users_hbm.at[pl.ds(wid * N_CH, N_CH)], idx_v)
    copies = [
        pltpu.async_copy(utab_hbm.at[idx_v.at[c]],
                         rows_v.at[pl.ds(c * IDX_CH, IDX_CH)], sem)
        for c in range(N_CH)
    ]

    @pl.when(wid == 0)
    def _():
      pltpu.sync_copy(nodes_hbm, nidx_v)
      pltpu.async_copy(itab_hbm.at[nidx_v], qrows_v, qsem).wait()
      pltpu.sync_copy(qrows_v, qj_hbm)

    for cp in copies:
      cp.wait()
    pltpu.sync_copy(rows_v, pt_hbm.at[pl.ds(wid * ROWS_PER_W, ROWS_PER_W)])

  return k(users2d, nodes_v, utab, itab)


def _tc_body(pt_ref, ratr_ref, segr_ref, qj_ref, r5_ref,
             g1_ref, g1b_ref, g2_ref, g2b_ref,
             a1_ref, a1b_ref, a2_ref, a2b_ref,
             a3_ref, a3b_ref, out_ref, macc, dacc, zacc,
             r1s, s1s, g1abf, g2bf, a1abf, a2bf):
  i = pl.program_id(0)
  f32 = jnp.float32
  bf16 = jnp.bfloat16

  @pl.when(i == 0)
  def _():
    macc[...] = jnp.full((B, 1), NEG, f32)
    dacc[...] = jnp.zeros((B, 1), f32)
    zacc[...] = jnp.zeros((B, D), f32)
    r1 = (lax.dot_general(r5_ref[...], g1_ref[:, D:], _NT,
                          preferred_element_type=f32) + g1b_ref[...])
    r1s[...] = jnp.concatenate(
        [r1, jnp.zeros((8 - NR, D), f32)], axis=0).astype(bf16)
    s1s[...] = (lax.dot_general(qj_ref[...], a1_ref[:, D:], _NT,
                                preferred_element_type=f32)
                + a1b_ref[...]).astype(bf16)
    g1abf[...] = g1_ref[:, :D].astype(bf16)
    g2bf[...] = g2_ref[...].astype(bf16)
    a1abf[...] = a1_ref[:, :D].astype(bf16)
    a2bf[...] = a2_ref[...].astype(bf16)

  pt = pt_ref[...].astype(bf16)
  ratr = ratr_ref[0, 0:1, :]
  segr = segr_ref[0, 0:1, :]

  rohT = (lax.broadcasted_iota(jnp.int32, (8, BLK), 0) == ratr).astype(bf16)
  sohT = lax.broadcasted_iota(jnp.int32, (B, BLK), 0) == segr
  sohTbf = sohT.astype(bf16)

  h = jnp.maximum(
      lax.dot_general(pt, g1abf[...], _NT, preferred_element_type=f32)
      + lax.dot_general(rohT, r1s[...], _TN, preferred_element_type=f32),
      0.0).astype(bf16)
  fjt = jnp.maximum(
      lax.dot_general(h, g2bf[...], _NT, preferred_element_type=f32)
      + g2b_ref[...], 0.0).astype(bf16)
  a = jnp.maximum(
      lax.dot_general(fjt, a1abf[...], _NT, preferred_element_type=f32)
      + lax.dot_general(sohTbf, s1s[...], _TN, preferred_element_type=f32),
      0.0).astype(bf16)
  a = jnp.maximum(
      lax.dot_general(a, a2bf[...], _NT, preferred_element_type=f32)
      + a2b_ref[...], 0.0).astype(bf16)
  s_row = lax.dot_general(a3_ref[...].astype(bf16), a, _NT,
                          preferred_element_type=f32) + a3b_ref[0, 0]

  bm = jnp.max(jnp.where(sohT, s_row, NEG), axis=1, keepdims=True)
  m_old = macc[...]
  m_new = jnp.maximum(m_old, bm)
  scale = jnp.exp(m_old - m_new)
  m_tok = jnp.max(jnp.where(sohT, m_new, NEG), axis=0, keepdims=True)
  e_row = jnp.exp(s_row - m_tok)
  w = jnp.where(sohT, e_row, 0.0)
  dacc[...] = dacc[...] * scale + jnp.sum(w, axis=1, keepdims=True)
  zacc[...] = zacc[...] * scale + jnp.dot(w.astype(bf16), fjt,
                                          preferred_element_type=f32)
  macc[...] = m_new

  @pl.when(i == NBLK - 1)
  def _():
    d = dacc[...]
    out_ref[...] = zacc[...] / jnp.where(d > 0, d, 1.0)


def _tc_call(pt, qj, flat_ratings, segment_ids, embed_r_w,
             g1_w, g1_b, g2_w, g2_b, a1_w, a1_b, a2_w, a2_b, a3_w, a3_b):
  full = lambda shape: pl.BlockSpec(shape, lambda i: (0,) * len(shape))
  return pl.pallas_call(
      _tc_body,
      grid=(NBLK,),
      in_specs=[
          pl.BlockSpec((BLK, D), lambda i: (i, 0)),
          pl.BlockSpec((1, 1, BLK), lambda i: (i, 0, 0)),
          pl.BlockSpec((1, 1, BLK), lambda i: (i, 0, 0)),
          full((B, D)),
          full((NR, D)),
          full((D, 2 * D)),
          full((1, D)),
          full((D, D)),
          full((1, D)),
          full((D, 2 * D)),
          full((1, D)),
          full((D, D)),
          full((1, D)),
          full((1, D)),
          full((1, 1)),
      ],
      out_specs=pl.BlockSpec((B, D), lambda i: (0, 0)),
      out_shape=jax.ShapeDtypeStruct((B, D), jnp.float32),
      scratch_shapes=[
          pltpu.VMEM((B, 1), jnp.float32),
          pltpu.VMEM((B, 1), jnp.float32),
          pltpu.VMEM((B, D), jnp.float32),
          pltpu.VMEM((8, D), jnp.bfloat16),
          pltpu.VMEM((B, D), jnp.bfloat16),
          pltpu.VMEM((D, D), jnp.bfloat16),
          pltpu.VMEM((D, D), jnp.bfloat16),
          pltpu.VMEM((D, D), jnp.bfloat16),
          pltpu.VMEM((D, D), jnp.bfloat16),
      ],
      compiler_params=pltpu.CompilerParams(
          dimension_semantics=("arbitrary",)),
  )(pt, flat_ratings.reshape(NBLK, 1, BLK),
    segment_ids.reshape(NBLK, 1, BLK), qj, embed_r_w,
    g1_w, g1_b.reshape(1, D), g2_w, g2_b.reshape(1, D),
    a1_w, a1_b.reshape(1, D), a2_w, a2_b.reshape(1, D),
    a3_w, a3_b.reshape(1, 1))


def kernel(nodes_v, flat_users, flat_ratings, segment_ids, embed_u_w,
           embed_i_w, embed_r_w, g1_w, g1_b, g2_w, g2_b, a1_w, a1_b,
           a2_w, a2_b, a3_w, a3_b):
  users2d = flat_users.reshape(T // IDX_CH, IDX_CH)
  pt, qj = _sc_gather(users2d, nodes_v, embed_u_w, embed_i_w)
  return _tc_call(pt, qj, flat_ratings, segment_ids, embed_r_w,
                  g1_w, g1_b, g2_w, g2_b, a1_w, a1_b, a2_w, a2_b, a3_w, a3_b)


# speedup vs baseline: 1.2504x; 1.0657x over previous
import functools

import jax
import jax.numpy as jnp
from jax import lax
from jax.experimental import pallas as pl
from jax.experimental.pallas import tpu as pltpu
from jax.experimental.pallas import tpu_sc as plsc

B = 16
T = 16384
D = 128
NR = 5
NW = 32
ROWS_PER_W = T // NW
IDX_CH = 128
N_CH = ROWS_PER_W // IDX_CH
BLK = 4096
NBLK = T // BLK
NH = 2
HBLK = BLK // NH
NEG = -3e38

_NT = (((1,), (1,)), ((), ()))
_TN = (((0,), (0,)), ((), ()))


def _sc_gather(users2d, nodes_v, utab, itab):
  mesh = plsc.VectorSubcoreMesh(core_axis_name="c", subcore_axis_name="s")

  @functools.partial(
      pl.kernel,
      mesh=mesh,
      out_type=(
          jax.ShapeDtypeStruct((T, D), jnp.float32),
          jax.ShapeDtypeStruct((B, D), jnp.float32),
      ),
      scratch_types=[
          pltpu.VMEM((N_CH, IDX_CH), jnp.int32),
          pltpu.VMEM((ROWS_PER_W, D), jnp.float32),
          pltpu.VMEM((B,), jnp.int32),
          pltpu.VMEM((B, D), jnp.float32),
          pltpu.SemaphoreType.DMA,
          pltpu.SemaphoreType.DMA,
      ],
  )
  def k(users_hbm, nodes_hbm, utab_hbm, itab_hbm, pt_hbm, qj_hbm,
        idx_v, rows_v, nidx_v, qrows_v, sem, qsem):
    wid = lax.axis_index("s") * 2 + lax.axis_index("c")
    pltpu.sync_copy(users_hbm.at[pl.ds(wid * N_CH, N_CH)], idx_v)
    copies = [
        pltpu.async_copy(utab_hbm.at[idx_v.at[c]],
                         rows_v.at[pl.ds(c * IDX_CH, IDX_CH)], sem)
        for c in range(N_CH)
    ]

    @pl.when(wid == 0)
    def _():
      pltpu.sync_copy(nodes_hbm, nidx_v)
      pltpu.async_copy(itab_hbm.at[nidx_v], qrows_v, qsem).wait()
      pltpu.sync_copy(qrows_v, qj_hbm)

    for cp in copies:
      cp.wait()
    pltpu.sync_copy(rows_v, pt_hbm.at[pl.ds(wid * ROWS_PER_W, ROWS_PER_W)])

  return k(users2d, nodes_v, utab, itab)


def _tc_body(pt_ref, ratr_ref, segr_ref, qj_ref, r5_ref,
             g1_ref, g1b_ref, g2_ref, g2b_ref,
             a1_ref, a1b_ref, a2_ref, a2b_ref,
             a3_ref, a3b_ref, out_ref, macc, dacc, zacc,
             r1s, s1s, g1abf, g2bf, a1abf, a2bf):
  i = pl.program_id(0)
  f32 = jnp.float32
  bf16 = jnp.bfloat16

  @pl.when(i == 0)
  def _():
    macc[...] = jnp.full((B, 1), NEG, f32)
    dacc[...] = jnp.zeros((B, 1), f32)
    zacc[...] = jnp.zeros((B, D), f32)
    r1 = (lax.dot_general(r5_ref[...], g1_ref[:, D:], _NT,
                          preferred_element_type=f32) + g1b_ref[...])
    r1s[...] = jnp.concatenate(
        [r1, jnp.zeros((8 - NR, D), f32)], axis=0).astype(bf16)
    s1s[...] = (lax.dot_general(qj_ref[...], a1_ref[:, D:], _NT,
                                preferred_element_type=f32)
                + a1b_ref[...]).astype(bf16)
    g1abf[...] = g1_ref[:, :D].astype(bf16)
    g2bf[...] = g2_ref[...].astype(bf16)
    a1abf[...] = a1_ref[:, :D].astype(bf16)
    a2bf[...] = a2_ref[...].astype(bf16)

  pt = pt_ref[...].astype(bf16)
  ratr = ratr_ref[0, 0:1, :]
  segr = segr_ref[0, 0:1, :]

  rohT = (lax.broadcasted_iota(jnp.int32, (8, BLK), 0) == ratr).astype(bf16)
  sohT = lax.broadcasted_iota(jnp.int32, (B, BLK), 0) == segr
  sohTbf = sohT.astype(bf16)

  h = jnp.maximum(
      lax.dot_general(pt, g1abf[...], _NT, preferred_element_type=f32)
      + lax.dot_general(rohT, r1s[...], _TN, preferred_element_type=f32),
      0.0).astype(bf16)
  fjt = jnp.maximum(
      lax.dot_general(h, g2bf[...], _NT, preferred_element_type=f32)
      + g2b_ref[...], 0.0).astype(bf16)
  a = jnp.maximum(
      lax.dot_general(fjt, a1abf[...], _NT, preferred_element_type=f32)
      + lax.dot_general(sohTbf, s1s[...], _TN, preferred_element_type=f32),
      0.0).astype(bf16)
  a = jnp.maximum(
      lax.dot_general(a, a2bf[...], _NT, preferred_element_type=f32)
      + a2b_ref[...], 0.0).astype(bf16)
  s_row = lax.dot_general(a3_ref[...].astype(bf16), a, _NT,
                          preferred_element_type=f32) + a3b_ref[0, 0]

  bm = jnp.max(jnp.where(sohT, s_row, NEG), axis=1, keepdims=True)
  m_old = macc[...]
  m_new = jnp.maximum(m_old, bm)
  scale = jnp.exp(m_old - m_new)
  m_tok = jnp.max(jnp.where(sohT, m_new, NEG), axis=0, keepdims=True)
  e_row = jnp.exp(s_row - m_tok)
  w = jnp.where(sohT, e_row, 0.0)
  dacc[...] = dacc[...] * scale + jnp.sum(w, axis=1, keepdims=True)
  zacc[...] = zacc[...] * scale + jnp.dot(w.astype(bf16), fjt,
                                          preferred_element_type=f32)
  macc[...] = m_new

  @pl.when(i == NBLK - 1)
  def _():
    d = dacc[...]
    out_ref[...] = zacc[...] / jnp.where(d > 0, d, 1.0)


def _tc_call(pt, qj, flat_ratings, segment_ids, embed_r_w,
             g1_w, g1_b, g2_w, g2_b, a1_w, a1_b, a2_w, a2_b, a3_w, a3_b):
  full = lambda shape: pl.BlockSpec(shape, lambda i: (0,) * len(shape))
  return pl.pallas_call(
      _tc_body,
      grid=(NBLK,),
      in_specs=[
          pl.BlockSpec((BLK, D), lambda i: (i, 0)),
          pl.BlockSpec((1, 1, BLK), lambda i: (i, 0, 0)),
          pl.BlockSpec((1, 1, BLK), lambda i: (i, 0, 0)),
          full((B, D)),
          full((NR, D)),
          full((D, 2 * D)),
          full((1, D)),
          full((D, D)),
          full((1, D)),
          full((D, 2 * D)),
          full((1, D)),
          full((D, D)),
          full((1, D)),
          full((1, D)),
          full((1, 1)),
      ],
      out_specs=pl.BlockSpec((B, D), lambda i: (0, 0)),
      out_shape=jax.ShapeDtypeStruct((B, D), jnp.float32),
      scratch_shapes=[
          pltpu.VMEM((B, 1), jnp.float32),
          pltpu.VMEM((B, 1), jnp.float32),
          pltpu.VMEM((B, D), jnp.float32),
          pltpu.VMEM((8, D), jnp.bfloat16),
          pltpu.VMEM((B, D), jnp.bfloat16),
          pltpu.VMEM((D, D), jnp.bfloat16),
          pltpu.VMEM((D, D), jnp.bfloat16),
          pltpu.VMEM((D, D), jnp.bfloat16),
          pltpu.VMEM((D, D), jnp.bfloat16),
      ],
      compiler_params=pltpu.CompilerParams(
          dimension_semantics=("arbitrary",)),
  )(pt, flat_ratings.reshape(NBLK, 1, BLK),
    segment_ids.reshape(NBLK, 1, BLK), qj, embed_r_w,
    g1_w, g1_b.reshape(1, D), g2_w, g2_b.reshape(1, D),
    a1_w, a1_b.reshape(1, D), a2_w, a2_b.reshape(1, D),
    a3_w, a3_b.reshape(1, 1))


def kernel(nodes_v, flat_users, flat_ratings, segment_ids, embed_u_w,
           embed_i_w, embed_r_w, g1_w, g1_b, g2_w, g2_b, a1_w, a1_b,
           a2_w, a2_b, a3_w, a3_b):
  users2d = flat_users.reshape(T // IDX_CH, IDX_CH)
  pt, qj = _sc_gather(users2d, nodes_v, embed_u_w, embed_i_w)
  return _tc_call(pt, qj, flat_ratings, segment_ids, embed_r_w,
                  g1_w, g1_b, g2_w, g2_b, a1_w, a1_b, a2_w, a2_b, a3_w, a3_b)


# speedup vs baseline: 1.2702x; 1.0158x over previous
import functools

import jax
import jax.numpy as jnp
from jax import lax
from jax.experimental import pallas as pl
from jax.experimental.pallas import tpu as pltpu
from jax.experimental.pallas import tpu_sc as plsc

B = 16
T = 16384
D = 128
NR = 5
NW = 32
ROWS_PER_W = T // NW
IDX_CH = 128
N_CH = ROWS_PER_W // IDX_CH
BLK = 8192
NBLK = T // BLK
NH = 2
HBLK = BLK // NH
NEG = -3e38

_NT = (((1,), (1,)), ((), ()))
_TN = (((0,), (0,)), ((), ()))


def _sc_gather(users2d, nodes_v, utab, itab):
  mesh = plsc.VectorSubcoreMesh(core_axis_name="c", subcore_axis_name="s")

  @functools.partial(
      pl.kernel,
      mesh=mesh,
      out_type=(
          jax.ShapeDtypeStruct((T, D), jnp.float32),
          jax.ShapeDtypeStruct((B, D), jnp.float32),
      ),
      scratch_types=[
          pltpu.VMEM((N_CH, IDX_CH), jnp.int32),
          pltpu.VMEM((ROWS_PER_W, D), jnp.float32),
          pltpu.VMEM((B,), jnp.int32),
          pltpu.VMEM((B, D), jnp.float32),
          pltpu.SemaphoreType.DMA,
          pltpu.SemaphoreType.DMA,
      ],
  )
  def k(users_hbm, nodes_hbm, utab_hbm, itab_hbm, pt_hbm, qj_hbm,
        idx_v, rows_v, nidx_v, qrows_v, sem, qsem):
    wid = lax.axis_index("s") * 2 + lax.axis_index("c")
    pltpu.sync_copy(users_hbm.at[pl.ds(wid * N_CH, N_CH)], idx_v)
    copies = [
        pltpu.async_copy(utab_hbm.at[idx_v.at[c]],
                         rows_v.at[pl.ds(c * IDX_CH, IDX_CH)], sem)
        for c in range(N_CH)
    ]

    @pl.when(wid == 0)
    def _():
      pltpu.sync_copy(nodes_hbm, nidx_v)
      pltpu.async_copy(itab_hbm.at[nidx_v], qrows_v, qsem).wait()
      pltpu.sync_copy(qrows_v, qj_hbm)

    for cp in copies:
      cp.wait()
    pltpu.sync_copy(rows_v, pt_hbm.at[pl.ds(wid * ROWS_PER_W, ROWS_PER_W)])

  return k(users2d, nodes_v, utab, itab)


def _tc_body(pt_ref, ratr_ref, segr_ref, qj_ref, r5_ref,
             g1_ref, g1b_ref, g2_ref, g2b_ref,
             a1_ref, a1b_ref, a2_ref, a2b_ref,
             a3_ref, a3b_ref, out_ref, macc, dacc, zacc,
             r1s, s1s, g1abf, g2bf, a1abf, a2bf):
  i = pl.program_id(0)
  f32 = jnp.float32
  bf16 = jnp.bfloat16

  @pl.when(i == 0)
  def _():
    macc[...] = jnp.full((B, 1), NEG, f32)
    dacc[...] = jnp.zeros((B, 1), f32)
    zacc[...] = jnp.zeros((B, D), f32)
    r1 = (lax.dot_general(r5_ref[...], g1_ref[:, D:], _NT,
                          preferred_element_type=f32) + g1b_ref[...])
    r1s[...] = jnp.concatenate(
        [r1, jnp.zeros((8 - NR, D), f32)], axis=0).astype(bf16)
    s1s[...] = (lax.dot_general(qj_ref[...], a1_ref[:, D:], _NT,
                                preferred_element_type=f32)
                + a1b_ref[...]).astype(bf16)
    g1abf[...] = g1_ref[:, :D].astype(bf16)
    g2bf[...] = g2_ref[...].astype(bf16)
    a1abf[...] = a1_ref[:, :D].astype(bf16)
    a2bf[...] = a2_ref[...].astype(bf16)

  pt = pt_ref[...].astype(bf16)
  ratr = ratr_ref[0, 0:1, :]
  segr = segr_ref[0, 0:1, :]

  rohT = (lax.broadcasted_iota(jnp.int32, (8, BLK), 0) == ratr).astype(bf16)
  sohT = lax.broadcasted_iota(jnp.int32, (B, BLK), 0) == segr
  sohTbf = sohT.astype(bf16)

  h = jnp.maximum(
      lax.dot_general(pt, g1abf[...], _NT, preferred_element_type=f32)
      + lax.dot_general(rohT, r1s[...], _TN, preferred_element_type=f32),
      0.0).astype(bf16)
  fjt = jnp.maximum(
      lax.dot_general(h, g2bf[...], _NT, preferred_element_type=f32)
      + g2b_ref[...], 0.0).astype(bf16)
  a = jnp.maximum(
      lax.dot_general(fjt, a1abf[...], _NT, preferred_element_type=f32)
      + lax.dot_general(sohTbf, s1s[...], _TN, preferred_element_type=f32),
      0.0).astype(bf16)
  a = jnp.maximum(
      lax.dot_general(a, a2bf[...], _NT, preferred_element_type=f32)
      + a2b_ref[...], 0.0).astype(bf16)
  s_row = lax.dot_general(a3_ref[...].astype(bf16), a, _NT,
                          preferred_element_type=f32) + a3b_ref[0, 0]

  bm = jnp.max(jnp.where(sohT, s_row, NEG), axis=1, keepdims=True)
  m_old = macc[...]
  m_new = jnp.maximum(m_old, bm)
  scale = jnp.exp(m_old - m_new)
  m_tok = jnp.max(jnp.where(sohT, m_new, NEG), axis=0, keepdims=True)
  e_row = jnp.exp(s_row - m_tok)
  w = jnp.where(sohT, e_row, 0.0)
  dacc[...] = dacc[...] * scale + jnp.sum(w, axis=1, keepdims=True)
  zacc[...] = zacc[...] * scale + jnp.dot(w.astype(bf16), fjt,
                                          preferred_element_type=f32)
  macc[...] = m_new

  @pl.when(i == NBLK - 1)
  def _():
    d = dacc[...]
    out_ref[...] = zacc[...] / jnp.where(d > 0, d, 1.0)


def _tc_call(pt, qj, flat_ratings, segment_ids, embed_r_w,
             g1_w, g1_b, g2_w, g2_b, a1_w, a1_b, a2_w, a2_b, a3_w, a3_b):
  full = lambda shape: pl.BlockSpec(shape, lambda i: (0,) * len(shape))
  return pl.pallas_call(
      _tc_body,
      grid=(NBLK,),
      in_specs=[
          pl.BlockSpec((BLK, D), lambda i: (i, 0)),
          pl.BlockSpec((1, 1, BLK), lambda i: (i, 0, 0)),
          pl.BlockSpec((1, 1, BLK), lambda i: (i, 0, 0)),
          full((B, D)),
          full((NR, D)),
          full((D, 2 * D)),
          full((1, D)),
          full((D, D)),
          full((1, D)),
          full((D, 2 * D)),
          full((1, D)),
          full((D, D)),
          full((1, D)),
          full((1, D)),
          full((1, 1)),
      ],
      out_specs=pl.BlockSpec((B, D), lambda i: (0, 0)),
      out_shape=jax.ShapeDtypeStruct((B, D), jnp.float32),
      scratch_shapes=[
          pltpu.VMEM((B, 1), jnp.float32),
          pltpu.VMEM((B, 1), jnp.float32),
          pltpu.VMEM((B, D), jnp.float32),
          pltpu.VMEM((8, D), jnp.bfloat16),
          pltpu.VMEM((B, D), jnp.bfloat16),
          pltpu.VMEM((D, D), jnp.bfloat16),
          pltpu.VMEM((D, D), jnp.bfloat16),
          pltpu.VMEM((D, D), jnp.bfloat16),
          pltpu.VMEM((D, D), jnp.bfloat16),
      ],
      compiler_params=pltpu.CompilerParams(
          dimension_semantics=("arbitrary",)),
  )(pt, flat_ratings.reshape(NBLK, 1, BLK),
    segment_ids.reshape(NBLK, 1, BLK), qj, embed_r_w,
    g1_w, g1_b.reshape(1, D), g2_w, g2_b.reshape(1, D),
    a1_w, a1_b.reshape(1, D), a2_w, a2_b.reshape(1, D),
    a3_w, a3_b.reshape(1, 1))


def kernel(nodes_v, flat_users, flat_ratings, segment_ids, embed_u_w,
           embed_i_w, embed_r_w, g1_w, g1_b, g2_w, g2_b, a1_w, a1_b,
           a2_w, a2_b, a3_w, a3_b):
  users2d = flat_users.reshape(T // IDX_CH, IDX_CH)
  pt, qj = _sc_gather(users2d, nodes_v, embed_u_w, embed_i_w)
  return _tc_call(pt, qj, flat_ratings, segment_ids, embed_r_w,
                  g1_w, g1_b, g2_w, g2_b, a1_w, a1_b, a2_w, a2_b, a3_w, a3_b)
